# baseline (speedup 1.0000x reference)
import functools

import jax
import jax.numpy as jnp
from jax import lax
from jax.experimental import pallas as pl
from jax.experimental.pallas import tpu as pltpu
from jax.experimental.pallas import tpu_sc as plsc

E = 320000
D = 128
V_SEG = 10000
NC = 2
NS = 16
NW = NC * NS
RW = E // NW
BI = 80
NBI = RW // BI
BL = 160
NBL = RW // BL
NBUF = 2
VCHUNK = 1000
NVT = V_SEG // VCHUNK


def _sc_partial_segment_sum(H, ids3, zrows):
    mesh = plsc.VectorSubcoreMesh(
        core_axis_name="c", subcore_axis_name="s",
        num_cores=NC, num_subcores=NS)

    @functools.partial(
        pl.kernel,
        out_type=jax.ShapeDtypeStruct((NC, V_SEG, D), jnp.float32),
        mesh=mesh,
        scratch_types=[
            pltpu.VMEM((NBUF, BL, D), jnp.float32),
            pltpu.VMEM((NBUF * (BL // BI), 1, BI), jnp.int32),
            pltpu.VMEM_SHARED((V_SEG, D), jnp.float32),
            [pltpu.SemaphoreType.DMA] * NBUF,
            [pltpu.SemaphoreType.DMA] * NBUF,
        ],
    )
    def k(h_hbm, ids_hbm, z_hbm, out_hbm, rows_v, ids_v, acc, lsems, ssems):
        c = lax.axis_index("c")
        s = lax.axis_index("s")
        wid = c * NS + s
        row_base = wid * RW
        vbase = s * VCHUNK

        @pl.when(s < NVT)
        def _zero():
            pltpu.sync_copy(z_hbm.at[pl.ds(vbase, VCHUNK), :],
                            acc.at[pl.ds(vbase, VCHUNK), :])

        plsc.subcore_barrier()
        SPB = BL // BI

        def load_desc(blk, b, rows):
            nid = rows // BI
            rdesc = pltpu.make_async_copy(
                h_hbm.at[pl.ds(row_base + blk * BL, rows), :],
                rows_v.at[b, pl.ds(0, rows), :], lsems[b])
            idesc = pltpu.make_async_copy(
                ids_hbm.at[wid, pl.ds(blk * SPB, nid), :, :],
                ids_v.at[pl.ds(b * SPB, nid)], lsems[b])

            class _Pair:
                def start(self):
                    rdesc.start()
                    idesc.start()

                def wait(self):
                    rdesc.wait()
                    idesc.wait()

            return _Pair()

        def scatter_descs(blk, b, nsub):
            del blk
            return [
                pltpu.make_async_copy(
                    rows_v.at[b, pl.ds(k * BI, BI), :],
                    acc.at[ids_v.at[b * SPB + k, 0]], ssems[b])
                for k in range(nsub)]

        def fire_scatter(blk, b, nsub=BL // BI):
            for d in scatter_descs(blk, b, nsub):
                d.start(add=True)

        def drain_scatter(blk, b, nsub=BL // BI):
            for d in scatter_descs(blk, b, nsub):
                d.wait()

        load_desc(0, 0, BL).start()

        @pl.loop(0, NBL, step=NBUF)
        def _ring(j):
            for b in range(NBUF):
                blk = j + b
                load_desc(blk, b, BL).wait()
                fire_scatter(blk, b)
                nb = (b + NBUF - 1) % NBUF

                @pl.when(blk >= 1)
                def _drain():
                    drain_scatter(blk - 1, nb)

                @pl.when(blk + NBUF - 1 < NBL)
                def _refill():
                    load_desc(blk + NBUF - 1, nb, BL).start()

                @pl.when(blk + NBUF - 1 == NBL)
                def _refill_tail():
                    load_desc(NBL, nb, BI).start()

        tb = NBL % NBUF
        load_desc(NBL, tb, BI).wait()
        fire_scatter(NBL, tb, nsub=1)
        drain_scatter(NBL - 1, (tb + NBUF - 1) % NBUF)
        drain_scatter(NBL, tb, nsub=1)
        plsc.subcore_barrier()

        @pl.when(s < NVT)
        def _write():
            pltpu.sync_copy(acc.at[pl.ds(vbase, VCHUNK), :],
                            out_hbm.at[c, pl.ds(vbase, VCHUNK), :])

    return k(H, ids3, zrows)


def _merge_partials(parts):
    BS = 1000

    def body(p_ref, o_ref):
        o_ref[...] = p_ref[0] + p_ref[1]

    return pl.pallas_call(
        body,
        grid=(V_SEG // BS,),
        in_specs=[pl.BlockSpec((NC, BS, D), lambda i: (0, i, 0))],
        out_specs=pl.BlockSpec((BS, D), lambda i: (i, 0)),
        out_shape=jax.ShapeDtypeStruct((V_SEG, D), jnp.float32),
    )(parts)


def kernel(H, X_neis, V):
    del V
    ids3 = X_neis.astype(jnp.int32).reshape(NW, NBI, 1, BI)
    zrows = jnp.zeros((V_SEG, D), jnp.float32)
    parts = _sc_partial_segment_sum(H, ids3, zrows)
    return _merge_partials(parts)


# speedup vs baseline: 1.1106x; 1.1106x over previous
import functools

import jax
import jax.numpy as jnp
from jax import lax
from jax.experimental import pallas as pl
from jax.experimental.pallas import tpu as pltpu
from jax.experimental.pallas import tpu_sc as plsc

E = 320000
D = 128
V_SEG = 10000
NC = 2
NS = 16
NW = NC * NS
RW = E // NW
BI = 40
NBI = RW // BI
BL = 40
NBL = RW // BL
NBUF = 8
VCHUNK = 1000
NVT = V_SEG // VCHUNK


def _sc_partial_segment_sum(H, ids3, zrows):
    mesh = plsc.VectorSubcoreMesh(
        core_axis_name="c", subcore_axis_name="s",
        num_cores=NC, num_subcores=NS)

    @functools.partial(
        pl.kernel,
        out_type=jax.ShapeDtypeStruct((NC, V_SEG, D), jnp.float32),
        mesh=mesh,
        scratch_types=[
            pltpu.VMEM((NBUF, BL, D), jnp.float32),
            pltpu.VMEM((NBUF * (BL // BI), 1, BI), jnp.int32),
            pltpu.VMEM_SHARED((V_SEG, D), jnp.float32),
            [pltpu.SemaphoreType.DMA] * NBUF,
            [pltpu.SemaphoreType.DMA] * NBUF,
        ],
    )
    def k(h_hbm, ids_hbm, z_hbm, out_hbm, rows_v, ids_v, acc, lsems, ssems):
        c = lax.axis_index("c")
        s = lax.axis_index("s")
        wid = c * NS + s
        row_base = wid * RW
        vbase = s * VCHUNK

        @pl.when(s < NVT)
        def _zero():
            pltpu.sync_copy(z_hbm.at[pl.ds(vbase, VCHUNK), :],
                            acc.at[pl.ds(vbase, VCHUNK), :])

        plsc.subcore_barrier()
        SPB = BL // BI

        def load_desc(blk, b, rows):
            nid = rows // BI
            rdesc = pltpu.make_async_copy(
                h_hbm.at[pl.ds(row_base + blk * BL, rows), :],
                rows_v.at[b, pl.ds(0, rows), :], lsems[b])
            idesc = pltpu.make_async_copy(
                ids_hbm.at[wid, pl.ds(blk * SPB, nid), :, :],
                ids_v.at[pl.ds(b * SPB, nid)], lsems[b])

            class _Pair:
                def start(self):
                    rdesc.start()
                    idesc.start()

                def wait(self):
                    rdesc.wait()
                    idesc.wait()

            return _Pair()

        def scatter_descs(blk, b, nsub):
            del blk
            return [
                pltpu.make_async_copy(
                    rows_v.at[b, pl.ds(k * BI, BI), :],
                    acc.at[ids_v.at[b * SPB + k, 0]], ssems[b])
                for k in range(nsub)]

        def fire_scatter(blk, b, nsub=BL // BI):
            for d in scatter_descs(blk, b, nsub):
                d.start(add=True)

        def drain_scatter(blk, b, nsub=BL // BI):
            for d in scatter_descs(blk, b, nsub):
                d.wait()

        for b in range(NBUF - 1):
            load_desc(b, b, BL).start()

        NMAIN = NBL - (NBL % NBUF)

        @pl.loop(0, NMAIN, step=NBUF)
        def _ring(j):
            for b in range(NBUF):
                blk = j + b
                load_desc(blk, b, BL).wait()
                fire_scatter(blk, b)
                nb = (b + NBUF - 1) % NBUF

                @pl.when(blk >= 1)
                def _drain():
                    drain_scatter(blk - 1, nb)

                @pl.when(blk + NBUF - 1 < NBL)
                def _refill():
                    load_desc(blk + NBUF - 1, nb, BL).start()

        for blk in range(NMAIN, NBL):
            b = blk % NBUF
            load_desc(blk, b, BL).wait()
            fire_scatter(blk, b)
            drain_scatter(blk - 1, (b + NBUF - 1) % NBUF)
        drain_scatter(NBL - 1, (NBL - 1) % NBUF)
        plsc.subcore_barrier()

        @pl.when(s < NVT)
        def _write():
            pltpu.sync_copy(acc.at[pl.ds(vbase, VCHUNK), :],
                            out_hbm.at[c, pl.ds(vbase, VCHUNK), :])

    return k(H, ids3, zrows)


def _merge_partials(parts):
    BS = 1000

    def body(p_ref, o_ref):
        o_ref[...] = p_ref[0] + p_ref[1]

    return pl.pallas_call(
        body,
        grid=(V_SEG // BS,),
        in_specs=[pl.BlockSpec((NC, BS, D), lambda i: (0, i, 0))],
        out_specs=pl.BlockSpec((BS, D), lambda i: (i, 0)),
        out_shape=jax.ShapeDtypeStruct((V_SEG, D), jnp.float32),
    )(parts)


def kernel(H, X_neis, V):
    del V
    ids3 = X_neis.astype(jnp.int32).reshape(NW, NBI, 1, BI)
    zrows = jnp.zeros((V_SEG, D), jnp.float32)
    parts = _sc_partial_segment_sum(H, ids3, zrows)
    return _merge_partials(parts)
